# baseline (speedup 1.0000x reference)
import functools

import jax
import jax.numpy as jnp
from jax import lax
from jax.experimental import pallas as pl
from jax.experimental.pallas import tpu as pltpu
from jax.experimental.pallas import tpu_sc as plsc

N = 10000
E = 320000
D = 128
R = 3
NQ = 4
DQ = D // NQ
NC, NS, LANES = 2, 16, 16
CH = 128
CHUNKS = 160
E_PAD = NS * CH * CHUNKS
ACC_TILE_ROWS = 1920
ACC_ROWS = NS * ACC_TILE_ROWS
DUMMY = R * N
OUT_TILE_ROWS = 1880
LAST_TILE_ROWS = R * N - (NS - 1) * OUT_TILE_ROWS
ZROWS = 128
BN_EPS = 1e-5
BLK = 2000
NBLK = N // BLK

_sc_mesh = plsc.VectorSubcoreMesh(core_axis_name="c", subcore_axis_name="s")


@functools.partial(
    pl.kernel,
    mesh=_sc_mesh,
    out_type=jax.ShapeDtypeStruct((R * N, D), jnp.float32),
    scratch_types=[
        pltpu.VMEM((CHUNKS, CH), jnp.int32),
        pltpu.VMEM((CHUNKS, CH), jnp.int32),
        pltpu.VMEM((CH, DQ), jnp.float32),
        pltpu.VMEM((CH, DQ), jnp.float32),
        pltpu.VMEM((CH, DQ), jnp.float32),
        pltpu.VMEM((CH, DQ), jnp.float32),
        pltpu.VMEM((ZROWS, DQ), jnp.float32),
        pltpu.VMEM_SHARED((ACC_ROWS, DQ), jnp.float32),
        pltpu.SemaphoreType.DMA,
        pltpu.SemaphoreType.DMA,
        pltpu.SemaphoreType.DMA,
        pltpu.SemaphoreType.DMA,
        pltpu.SemaphoreType.DMA,
        pltpu.SemaphoreType.DMA,
        pltpu.SemaphoreType.DMA,
        pltpu.SemaphoreType.DMA,
        pltpu.SemaphoreType.DMA,
    ],
    compiler_params=pltpu.CompilerParams(use_tc_tiling_on_sc=False),
)
def _sc_agg(gidx_hbm, sidx_hbm, vq_hbm, out_hbm, gix, six,
            rows0, rows1, rows2, rows3, zb, acc, sem,
            gs0, gs1, gs2, gs3, ss0, ss1, ss2, ss3):
    c = lax.axis_index("c")
    t = lax.axis_index("s")

    cp = pltpu.async_copy(sidx_hbm.at[t], six, sem)
    zv = jnp.zeros((LANES,), jnp.float32)

    @pl.loop(0, ZROWS)
    def _(i):
        for k in range(DQ // LANES):
            zb.at[i, pl.ds(k * LANES, LANES)][...] = zv

    cp.wait()

    for p in range(NQ // NC):
        q = c + 2 * p
        cpg = pltpu.async_copy(gidx_hbm.at[(c * 2 + p) * NS + t], gix, sem)
        base = t * ACC_TILE_ROWS
        for k in range(ACC_TILE_ROWS // ZROWS):
            pltpu.sync_copy(zb, acc.at[pl.ds(base + k * ZROWS, ZROWS)])
        cpg.wait()
        plsc.subcore_barrier()

        bufs = (rows0, rows1, rows2, rows3)
        gsems = (gs0, gs1, gs2, gs3)
        ssems = (ss0, ss1, ss2, ss3)
        for b in range(4):
            pltpu.async_copy(vq_hbm.at[gix.at[b]], bufs[b], gsems[b])

        @pl.loop(0, CHUNKS // 4)
        def _(k):
            j = 4 * k
            for b in range(4):
                pltpu.make_async_copy(vq_hbm.at[gix.at[j + b]], bufs[b],
                                      gsems[b]).wait()
                pltpu.sync_copy(bufs[b], acc.at[six.at[j + b]], add=True)

                @pl.when(k < CHUNKS // 4 - 1)
                def _():
                    pltpu.async_copy(vq_hbm.at[gix.at[j + b + 4]], bufs[b],
                                     gsems[b])

        plsc.subcore_barrier()
        ob = t * OUT_TILE_ROWS

        @pl.when(t < NS - 1)
        def _():
            pltpu.sync_copy(acc.at[pl.ds(ob, OUT_TILE_ROWS)],
                            out_hbm.at[pl.ds(ob, OUT_TILE_ROWS),
                                       pl.ds(q * DQ, DQ)])

        @pl.when(t == NS - 1)
        def _():
            pltpu.sync_copy(acc.at[pl.ds(ob, LAST_TILE_ROWS)],
                            out_hbm.at[pl.ds(ob, LAST_TILE_ROWS),
                                       pl.ds(q * DQ, DQ)])

        plsc.subcore_barrier()


def _tc1_body(x_ref, agg_ref, w1aT, w2aT, w1bT, w2bT,
              b1a, b2a, b1b, b2b, o3, s_ref, ss_ref):
    i = pl.program_id(0)
    x = x_ref[...]
    a = agg_ref[...]
    asum = a[0] + a[1] + a[2]
    s = jnp.zeros((1, D), jnp.float32)
    ss = jnp.zeros((1, D), jnp.float32)
    f32 = jnp.float32
    for r in range(R):
        h1 = x + a[r]
        h2 = x + (asum - a[r])
        g1 = jnp.maximum(jnp.dot(h1, w1aT[...], preferred_element_type=f32)
                         + b1a[...], 0.0)
        v1 = jnp.dot(g1, w2aT[...], preferred_element_type=f32) + b2a[...]
        g2 = jnp.maximum(jnp.dot(h2, w1bT[...], preferred_element_type=f32)
                         + b1b[...], 0.0)
        v2 = jnp.dot(g2, w2bT[...], preferred_element_type=f32) + b2b[...]
        o = v1 + v2
        o3[r] = o
        s = s + jnp.sum(o, axis=0, keepdims=True)
        ss = ss + jnp.sum(o * o, axis=0, keepdims=True)

    @pl.when(i == 0)
    def _():
        s_ref[...] = s
        ss_ref[...] = ss

    @pl.when(i > 0)
    def _():
        s_ref[...] += s
        ss_ref[...] += ss


_tc1 = pl.pallas_call(
    _tc1_body,
    grid=(NBLK,),
    in_specs=[
        pl.BlockSpec((BLK, D), lambda i: (i, 0)),
        pl.BlockSpec((R, BLK, D), lambda i: (0, i, 0)),
        pl.BlockSpec((D, D), lambda i: (0, 0)),
        pl.BlockSpec((D, D), lambda i: (0, 0)),
        pl.BlockSpec((D, D), lambda i: (0, 0)),
        pl.BlockSpec((D, D), lambda i: (0, 0)),
        pl.BlockSpec((1, D), lambda i: (0, 0)),
        pl.BlockSpec((1, D), lambda i: (0, 0)),
        pl.BlockSpec((1, D), lambda i: (0, 0)),
        pl.BlockSpec((1, D), lambda i: (0, 0)),
    ],
    out_specs=[
        pl.BlockSpec((R, BLK, D), lambda i: (0, i, 0)),
        pl.BlockSpec((1, D), lambda i: (0, 0)),
        pl.BlockSpec((1, D), lambda i: (0, 0)),
    ],
    out_shape=[
        jax.ShapeDtypeStruct((R, N, D), jnp.float32),
        jax.ShapeDtypeStruct((1, D), jnp.float32),
        jax.ShapeDtypeStruct((1, D), jnp.float32),
    ],
)


def _tc2_body(o3, s_ref, ss_ref, g_ref, b_ref, out):
    inv = 1.0 / float(R * N)
    mean = s_ref[...] * inv
    var = ss_ref[...] * inv - mean * mean
    scale = g_ref[...] * lax.rsqrt(var + BN_EPS)
    shift = b_ref[...] - mean * scale
    for r in range(R):
        out[:, r, :] = o3[r] * scale + shift


_tc2 = pl.pallas_call(
    _tc2_body,
    grid=(NBLK,),
    in_specs=[
        pl.BlockSpec((R, BLK, D), lambda i: (0, i, 0)),
        pl.BlockSpec((1, D), lambda i: (0, 0)),
        pl.BlockSpec((1, D), lambda i: (0, 0)),
        pl.BlockSpec((1, D), lambda i: (0, 0)),
        pl.BlockSpec((1, D), lambda i: (0, 0)),
    ],
    out_specs=pl.BlockSpec((BLK, R, D), lambda i: (i, 0, 0)),
    out_shape=jax.ShapeDtypeStruct((N, R, D), jnp.float32),
)


def kernel(vfts, adjs, rels, W1a, b1a, W2a, b2a, W1b, b1b, W2b, b2b, gamma, beta):
    src = adjs[0]
    dst = adjs[1]
    pad = E_PAD - E
    sidx = rels * N + dst
    sidx_p = jnp.concatenate([sidx, jnp.full((pad,), DUMMY, jnp.int32)])
    src_p = jnp.concatenate([src, jnp.zeros((pad,), jnp.int32)])
    base = src_p * NQ
    off = jnp.array([0, 2, 1, 3], jnp.int32)
    gidx = (base[None, :] + off[:, None]).reshape(NC * 2 * NS, CHUNKS, CH)
    sidx_slab = sidx_p.reshape(NS, CHUNKS, CH)
    vq = vfts.reshape(N * NQ, DQ)

    agg = _sc_agg(gidx, sidx_slab, vq)
    agg3 = agg.reshape(R, N, D)

    o3, s, ss = _tc1(
        vfts, agg3, W1a.T, W2a.T, W1b.T, W2b.T,
        b1a.reshape(1, D), b2a.reshape(1, D),
        b1b.reshape(1, D), b2b.reshape(1, D),
    )
    return _tc2(o3, s, ss, gamma.reshape(1, D), beta.reshape(1, D))


# speedup vs baseline: 1.1897x; 1.1897x over previous
import functools

import jax
import jax.numpy as jnp
from jax import lax
from jax.experimental import pallas as pl
from jax.experimental.pallas import tpu as pltpu
from jax.experimental.pallas import tpu_sc as plsc

N = 10000
E = 320000
D = 128
R = 3
NQ = 4
DQ = D // NQ
NC, NS, LANES = 2, 16, 16
CH = 128
CHUNKS = 158
E_PAD = NS * CH * CHUNKS
ACC_TILE_ROWS = 1920
ACC_ROWS = NS * ACC_TILE_ROWS
DUMMY = R * N
OUT_TILE_ROWS = 1880
LAST_TILE_ROWS = R * N - (NS - 1) * OUT_TILE_ROWS
ZROWS = 128
BN_EPS = 1e-5
BLK = 2000
NBLK = N // BLK

_sc_mesh = plsc.VectorSubcoreMesh(core_axis_name="c", subcore_axis_name="s")


@functools.partial(
    pl.kernel,
    mesh=_sc_mesh,
    out_type=jax.ShapeDtypeStruct((R * N, D), jnp.float32),
    scratch_types=[
        pltpu.VMEM((CHUNKS, CH), jnp.int32),
        pltpu.VMEM((CHUNKS, CH), jnp.int32),
        pltpu.VMEM((CH, DQ), jnp.float32),
        pltpu.VMEM((CH, DQ), jnp.float32),
        pltpu.VMEM((CH, DQ), jnp.float32),
        pltpu.VMEM((CH, DQ), jnp.float32),
        pltpu.VMEM((ZROWS, DQ), jnp.float32),
        pltpu.VMEM_SHARED((ACC_ROWS, DQ), jnp.float32),
        pltpu.SemaphoreType.DMA,
        pltpu.SemaphoreType.DMA,
        pltpu.SemaphoreType.DMA,
        pltpu.SemaphoreType.DMA,
        pltpu.SemaphoreType.DMA,
        pltpu.SemaphoreType.DMA,
        pltpu.SemaphoreType.DMA,
        pltpu.SemaphoreType.DMA,
        pltpu.SemaphoreType.DMA,
    ],
    compiler_params=pltpu.CompilerParams(use_tc_tiling_on_sc=False),
)
def _sc_agg(gidx_hbm, sidx_hbm, vq_hbm, out_hbm, gix, six,
            rows0, rows1, rows2, rows3, zb, acc, sem,
            gs0, gs1, gs2, gs3, ss0, ss1, ss2, ss3):
    c = lax.axis_index("c")
    t = lax.axis_index("s")

    cp = pltpu.async_copy(sidx_hbm.at[t], six, sem)
    zv = jnp.zeros((LANES,), jnp.float32)

    @pl.loop(0, ZROWS)
    def _(i):
        for k in range(DQ // LANES):
            zb.at[i, pl.ds(k * LANES, LANES)][...] = zv

    cp.wait()

    for p in range(NQ // NC):
        q = c + 2 * p
        cpg = pltpu.async_copy(gidx_hbm.at[(c * 2 + p) * NS + t], gix, sem)
        base = t * ACC_TILE_ROWS
        for k in range(ACC_TILE_ROWS // ZROWS):
            pltpu.sync_copy(zb, acc.at[pl.ds(base + k * ZROWS, ZROWS)])
        cpg.wait()
        plsc.subcore_barrier()

        bufs = (rows0, rows1)
        gsems = (gs0, gs1)
        ssems = (ss0, ss1)
        for b in range(2):
            pltpu.async_copy(vq_hbm.at[gix.at[b]], bufs[b], gsems[b])

        @pl.loop(0, CHUNKS // 2)
        def _(k):
            j = 2 * k
            scats = []
            for b in range(2):
                pltpu.make_async_copy(vq_hbm.at[gix.at[j + b]], bufs[b],
                                      gsems[b]).wait()
                scats.append(pltpu.async_copy(bufs[b], acc.at[six.at[j + b]],
                                              ssems[b], add=True))
            for b in range(2):
                scats[b].wait()

                @pl.when(k < CHUNKS // 2 - 1)
                def _():
                    pltpu.async_copy(vq_hbm.at[gix.at[j + b + 2]], bufs[b],
                                     gsems[b])

        plsc.subcore_barrier()
        ob = t * OUT_TILE_ROWS

        @pl.when(t < NS - 1)
        def _():
            pltpu.sync_copy(acc.at[pl.ds(ob, OUT_TILE_ROWS)],
                            out_hbm.at[pl.ds(ob, OUT_TILE_ROWS),
                                       pl.ds(q * DQ, DQ)])

        @pl.when(t == NS - 1)
        def _():
            pltpu.sync_copy(acc.at[pl.ds(ob, LAST_TILE_ROWS)],
                            out_hbm.at[pl.ds(ob, LAST_TILE_ROWS),
                                       pl.ds(q * DQ, DQ)])

        plsc.subcore_barrier()


def _tc1_body(x_ref, agg_ref, w1aT, w2aT, w1bT, w2bT,
              b1a, b2a, b1b, b2b, o3, s_ref, ss_ref):
    i = pl.program_id(0)
    x = x_ref[...]
    a = agg_ref[...]
    asum = a[0] + a[1] + a[2]
    s = jnp.zeros((1, D), jnp.float32)
    ss = jnp.zeros((1, D), jnp.float32)
    f32 = jnp.float32
    for r in range(R):
        h1 = x + a[r]
        h2 = x + (asum - a[r])
        g1 = jnp.maximum(jnp.dot(h1, w1aT[...], preferred_element_type=f32)
                         + b1a[...], 0.0)
        v1 = jnp.dot(g1, w2aT[...], preferred_element_type=f32) + b2a[...]
        g2 = jnp.maximum(jnp.dot(h2, w1bT[...], preferred_element_type=f32)
                         + b1b[...], 0.0)
        v2 = jnp.dot(g2, w2bT[...], preferred_element_type=f32) + b2b[...]
        o = v1 + v2
        o3[r] = o
        s = s + jnp.sum(o, axis=0, keepdims=True)
        ss = ss + jnp.sum(o * o, axis=0, keepdims=True)

    @pl.when(i == 0)
    def _():
        s_ref[...] = s
        ss_ref[...] = ss

    @pl.when(i > 0)
    def _():
        s_ref[...] += s
        ss_ref[...] += ss


_tc1 = pl.pallas_call(
    _tc1_body,
    grid=(NBLK,),
    in_specs=[
        pl.BlockSpec((BLK, D), lambda i: (i, 0)),
        pl.BlockSpec((R, BLK, D), lambda i: (0, i, 0)),
        pl.BlockSpec((D, D), lambda i: (0, 0)),
        pl.BlockSpec((D, D), lambda i: (0, 0)),
        pl.BlockSpec((D, D), lambda i: (0, 0)),
        pl.BlockSpec((D, D), lambda i: (0, 0)),
        pl.BlockSpec((1, D), lambda i: (0, 0)),
        pl.BlockSpec((1, D), lambda i: (0, 0)),
        pl.BlockSpec((1, D), lambda i: (0, 0)),
        pl.BlockSpec((1, D), lambda i: (0, 0)),
    ],
    out_specs=[
        pl.BlockSpec((R, BLK, D), lambda i: (0, i, 0)),
        pl.BlockSpec((1, D), lambda i: (0, 0)),
        pl.BlockSpec((1, D), lambda i: (0, 0)),
    ],
    out_shape=[
        jax.ShapeDtypeStruct((R, N, D), jnp.float32),
        jax.ShapeDtypeStruct((1, D), jnp.float32),
        jax.ShapeDtypeStruct((1, D), jnp.float32),
    ],
)


def _tc2_body(o3, s_ref, ss_ref, g_ref, b_ref, out):
    inv = 1.0 / float(R * N)
    mean = s_ref[...] * inv
    var = ss_ref[...] * inv - mean * mean
    scale = g_ref[...] * lax.rsqrt(var + BN_EPS)
    shift = b_ref[...] - mean * scale
    for r in range(R):
        out[:, r, :] = o3[r] * scale + shift


_tc2 = pl.pallas_call(
    _tc2_body,
    grid=(NBLK,),
    in_specs=[
        pl.BlockSpec((R, BLK, D), lambda i: (0, i, 0)),
        pl.BlockSpec((1, D), lambda i: (0, 0)),
        pl.BlockSpec((1, D), lambda i: (0, 0)),
        pl.BlockSpec((1, D), lambda i: (0, 0)),
        pl.BlockSpec((1, D), lambda i: (0, 0)),
    ],
    out_specs=pl.BlockSpec((BLK, R, D), lambda i: (i, 0, 0)),
    out_shape=jax.ShapeDtypeStruct((N, R, D), jnp.float32),
)


def kernel(vfts, adjs, rels, W1a, b1a, W2a, b2a, W1b, b1b, W2b, b2b, gamma, beta):
    src = adjs[0]
    dst = adjs[1]
    pad = E_PAD - E
    sidx = rels * N + dst
    sidx_p = jnp.concatenate([sidx, jnp.full((pad,), DUMMY, jnp.int32)])
    src_p = jnp.concatenate([src, jnp.zeros((pad,), jnp.int32)])
    base = src_p * NQ
    off = jnp.array([0, 2, 1, 3], jnp.int32)
    gidx = (base[None, :] + off[:, None]).reshape(NC * 2 * NS, CHUNKS, CH)
    sidx_slab = sidx_p.reshape(NS, CHUNKS, CH)
    vq = vfts.reshape(N * NQ, DQ)

    agg = _sc_agg(gidx, sidx_slab, vq)
    agg3 = agg.reshape(R, N, D)

    o3, s, ss = _tc1(
        vfts, agg3, W1a.T, W2a.T, W1b.T, W2b.T,
        b1a.reshape(1, D), b2a.reshape(1, D),
        b1b.reshape(1, D), b2b.reshape(1, D),
    )
    return _tc2(o3, s, ss, gamma.reshape(1, D), beta.reshape(1, D))


# speedup vs baseline: 1.2397x; 1.0420x over previous
import functools

import jax
import jax.numpy as jnp
from jax import lax
from jax.experimental import pallas as pl
from jax.experimental.pallas import tpu as pltpu
from jax.experimental.pallas import tpu_sc as plsc

N = 10000
E = 320000
D = 128
R = 3
NQ = 4
DQ = D // NQ
NC, NS, LANES = 2, 16, 16
CH = 128
CHUNKS = 158
E_PAD = NS * CH * CHUNKS
ACC_TILE_ROWS = 1920
ACC_ROWS = NS * ACC_TILE_ROWS
DUMMY = R * N
OUT_TILE_ROWS = 1880
LAST_TILE_ROWS = R * N - (NS - 1) * OUT_TILE_ROWS
ZROWS = 480
BN_EPS = 1e-5
BLK = 2000
NBLK = N // BLK

_sc_mesh = plsc.VectorSubcoreMesh(core_axis_name="c", subcore_axis_name="s")


@functools.partial(
    pl.kernel,
    mesh=_sc_mesh,
    out_type=jax.ShapeDtypeStruct((R * N, D), jnp.float32),
    scratch_types=[
        pltpu.VMEM((CHUNKS, CH), jnp.int32),
        pltpu.VMEM((CHUNKS, CH), jnp.int32),
        pltpu.VMEM((CH, DQ), jnp.float32),
        pltpu.VMEM((CH, DQ), jnp.float32),
        pltpu.VMEM((ZROWS, DQ), jnp.float32),
        pltpu.VMEM_SHARED((ACC_ROWS, DQ), jnp.float32),
        pltpu.SemaphoreType.DMA,
        pltpu.SemaphoreType.DMA,
        pltpu.SemaphoreType.DMA,
    ],
    compiler_params=pltpu.CompilerParams(use_tc_tiling_on_sc=False),
)
def _sc_agg(gidx_hbm, sidx_hbm, vq_hbm, out_hbm, gix, six,
            rows0, rows1, zb, acc, sem, gs0, gs1):
    c = lax.axis_index("c")
    t = lax.axis_index("s")
    bufs = (rows0, rows1)
    gsems = (gs0, gs1)

    cp = pltpu.async_copy(sidx_hbm.at[t], six, sem)
    zv = jnp.zeros((LANES,), jnp.float32)

    @pl.loop(0, ZROWS)
    def _(i):
        for k in range(DQ // LANES):
            zb.at[i, pl.ds(k * LANES, LANES)][...] = zv

    cp.wait()

    for p in range(NQ // NC):
        q = c + 2 * p
        pltpu.sync_copy(gidx_hbm.at[(c * 2 + p) * NS + t], gix)
        for b in range(2):
            pltpu.async_copy(vq_hbm.at[gix.at[b]], bufs[b], gsems[b])
        base = t * ACC_TILE_ROWS
        for k in range(ACC_TILE_ROWS // ZROWS):
            pltpu.sync_copy(zb, acc.at[pl.ds(base + k * ZROWS, ZROWS)])
        plsc.subcore_barrier()

        @pl.loop(0, CHUNKS // 2)
        def _(k):
            j = 2 * k
            for b in range(2):
                pltpu.make_async_copy(vq_hbm.at[gix.at[j + b]], bufs[b],
                                      gsems[b]).wait()
                pltpu.sync_copy(bufs[b], acc.at[six.at[j + b]], add=True)

                @pl.when(k < CHUNKS // 2 - 1)
                def _():
                    pltpu.async_copy(vq_hbm.at[gix.at[j + b + 2]], bufs[b],
                                     gsems[b])

        plsc.subcore_barrier()
        ob = t * OUT_TILE_ROWS

        @pl.when(t < NS - 1)
        def _():
            pltpu.sync_copy(acc.at[pl.ds(ob, OUT_TILE_ROWS)],
                            out_hbm.at[pl.ds(ob, OUT_TILE_ROWS),
                                       pl.ds(q * DQ, DQ)])

        @pl.when(t == NS - 1)
        def _():
            pltpu.sync_copy(acc.at[pl.ds(ob, LAST_TILE_ROWS)],
                            out_hbm.at[pl.ds(ob, LAST_TILE_ROWS),
                                       pl.ds(q * DQ, DQ)])

        plsc.subcore_barrier()


def _tc1_body(x_ref, agg_ref, w1aT, w2aT, w1bT, w2bT,
              b1a, b2a, b1b, b2b, o3, s_ref, ss_ref):
    i = pl.program_id(0)
    x = x_ref[...]
    a = agg_ref[...]
    asum = a[0] + a[1] + a[2]
    s = jnp.zeros((1, D), jnp.float32)
    ss = jnp.zeros((1, D), jnp.float32)
    f32 = jnp.float32
    for r in range(R):
        h1 = x + a[r]
        h2 = x + (asum - a[r])
        g1 = jnp.maximum(jnp.dot(h1, w1aT[...], preferred_element_type=f32)
                         + b1a[...], 0.0)
        v1 = jnp.dot(g1, w2aT[...], preferred_element_type=f32) + b2a[...]
        g2 = jnp.maximum(jnp.dot(h2, w1bT[...], preferred_element_type=f32)
                         + b1b[...], 0.0)
        v2 = jnp.dot(g2, w2bT[...], preferred_element_type=f32) + b2b[...]
        o = v1 + v2
        o3[r] = o
        s = s + jnp.sum(o, axis=0, keepdims=True)
        ss = ss + jnp.sum(o * o, axis=0, keepdims=True)

    @pl.when(i == 0)
    def _():
        s_ref[...] = s
        ss_ref[...] = ss

    @pl.when(i > 0)
    def _():
        s_ref[...] += s
        ss_ref[...] += ss


_tc1 = pl.pallas_call(
    _tc1_body,
    grid=(NBLK,),
    in_specs=[
        pl.BlockSpec((BLK, D), lambda i: (i, 0)),
        pl.BlockSpec((R, BLK, D), lambda i: (0, i, 0)),
        pl.BlockSpec((D, D), lambda i: (0, 0)),
        pl.BlockSpec((D, D), lambda i: (0, 0)),
        pl.BlockSpec((D, D), lambda i: (0, 0)),
        pl.BlockSpec((D, D), lambda i: (0, 0)),
        pl.BlockSpec((1, D), lambda i: (0, 0)),
        pl.BlockSpec((1, D), lambda i: (0, 0)),
        pl.BlockSpec((1, D), lambda i: (0, 0)),
        pl.BlockSpec((1, D), lambda i: (0, 0)),
    ],
    out_specs=[
        pl.BlockSpec((R, BLK, D), lambda i: (0, i, 0)),
        pl.BlockSpec((1, D), lambda i: (0, 0)),
        pl.BlockSpec((1, D), lambda i: (0, 0)),
    ],
    out_shape=[
        jax.ShapeDtypeStruct((R, N, D), jnp.float32),
        jax.ShapeDtypeStruct((1, D), jnp.float32),
        jax.ShapeDtypeStruct((1, D), jnp.float32),
    ],
)


def _tc2_body(o3, s_ref, ss_ref, g_ref, b_ref, out):
    inv = 1.0 / float(R * N)
    mean = s_ref[...] * inv
    var = ss_ref[...] * inv - mean * mean
    scale = g_ref[...] * lax.rsqrt(var + BN_EPS)
    shift = b_ref[...] - mean * scale
    for r in range(R):
        out[:, r, :] = o3[r] * scale + shift


_tc2 = pl.pallas_call(
    _tc2_body,
    grid=(NBLK,),
    in_specs=[
        pl.BlockSpec((R, BLK, D), lambda i: (0, i, 0)),
        pl.BlockSpec((1, D), lambda i: (0, 0)),
        pl.BlockSpec((1, D), lambda i: (0, 0)),
        pl.BlockSpec((1, D), lambda i: (0, 0)),
        pl.BlockSpec((1, D), lambda i: (0, 0)),
    ],
    out_specs=pl.BlockSpec((BLK, R, D), lambda i: (i, 0, 0)),
    out_shape=jax.ShapeDtypeStruct((N, R, D), jnp.float32),
)


def kernel(vfts, adjs, rels, W1a, b1a, W2a, b2a, W1b, b1b, W2b, b2b, gamma, beta):
    src = adjs[0]
    dst = adjs[1]
    pad = E_PAD - E
    sidx = rels * N + dst
    sidx_p = jnp.concatenate([sidx, jnp.full((pad,), DUMMY, jnp.int32)])
    src_p = jnp.concatenate([src, jnp.zeros((pad,), jnp.int32)])
    base = src_p * NQ
    off = jnp.array([0, 2, 1, 3], jnp.int32)
    gidx = (base[None, :] + off[:, None]).reshape(NC * 2 * NS, CHUNKS, CH)
    sidx_slab = sidx_p.reshape(NS, CHUNKS, CH)
    vq = vfts.reshape(N * NQ, DQ)

    agg = _sc_agg(gidx, sidx_slab, vq)
    agg3 = agg.reshape(R, N, D)

    o3, s, ss = _tc1(
        vfts, agg3, W1a.T, W2a.T, W1b.T, W2b.T,
        b1a.reshape(1, D), b2a.reshape(1, D),
        b1b.reshape(1, D), b2b.reshape(1, D),
    )
    return _tc2(o3, s, ss, gamma.reshape(1, D), beta.reshape(1, D))


# speedup vs baseline: 1.3016x; 1.0500x over previous
import functools

import jax
import jax.numpy as jnp
from jax import lax
from jax.experimental import pallas as pl
from jax.experimental.pallas import tpu as pltpu
from jax.experimental.pallas import tpu_sc as plsc

N = 10000
E = 320000
D = 128
R = 3
NQ = 4
DQ = D // NQ
NC, NS, LANES = 2, 16, 16
CH = 128
CHUNKS = 158
E_PAD = NS * CH * CHUNKS
ACC_TILE_ROWS = 1920
ACC_ROWS = NS * ACC_TILE_ROWS
DUMMY = R * N
OUT_TILE_ROWS = 1880
LAST_TILE_ROWS = R * N - (NS - 1) * OUT_TILE_ROWS
ZROWS = 480
BN_EPS = 1e-5
BLK = 2000
NBLK = N // BLK

_sc_mesh = plsc.VectorSubcoreMesh(core_axis_name="c", subcore_axis_name="s")


@functools.partial(
    pl.kernel,
    mesh=_sc_mesh,
    out_type=jax.ShapeDtypeStruct((R * N, D), jnp.float32),
    scratch_types=[
        pltpu.VMEM((CHUNKS, CH), jnp.int32),
        pltpu.VMEM((CHUNKS, CH), jnp.int32),
        pltpu.VMEM((CH, DQ), jnp.float32),
        pltpu.VMEM((CH, DQ), jnp.float32),
        pltpu.VMEM((ZROWS, DQ), jnp.float32),
        pltpu.VMEM_SHARED((ACC_ROWS, DQ), jnp.float32),
        pltpu.SemaphoreType.DMA,
        pltpu.SemaphoreType.DMA,
        pltpu.SemaphoreType.DMA,
    ],
    compiler_params=pltpu.CompilerParams(use_tc_tiling_on_sc=False),
)
def _sc_agg(gidx_hbm, sidx_hbm, vq_hbm, out_hbm, gix, six,
            rows0, rows1, zb, acc, sem, gs0, gs1):
    c = lax.axis_index("c")
    t = lax.axis_index("s")
    bufs = (rows0, rows1)
    gsems = (gs0, gs1)

    cp = pltpu.async_copy(sidx_hbm.at[t], six, sem)
    zv = jnp.zeros((LANES,), jnp.float32)

    @pl.loop(0, ZROWS)
    def _(i):
        for k in range(DQ // LANES):
            zb.at[i, pl.ds(k * LANES, LANES)][...] = zv

    cp.wait()

    for p in range(NQ // NC):
        q = c + 2 * p
        pltpu.sync_copy(gidx_hbm.at[(c * 2 + p) * NS + t], gix)
        for b in range(2):
            pltpu.async_copy(vq_hbm.at[gix.at[b]], bufs[b], gsems[b])
        base = t * ACC_TILE_ROWS
        for k in range(ACC_TILE_ROWS // ZROWS):
            pltpu.sync_copy(zb, acc.at[pl.ds(base + k * ZROWS, ZROWS)])
        plsc.subcore_barrier()

        @pl.loop(0, CHUNKS // 2)
        def _(k):
            j = 2 * k
            for b in range(2):
                pltpu.make_async_copy(vq_hbm.at[gix.at[j + b]], bufs[b],
                                      gsems[b]).wait()
                pltpu.sync_copy(bufs[b], acc.at[six.at[j + b]], add=True)

                @pl.when(k < CHUNKS // 2 - 1)
                def _():
                    pltpu.async_copy(vq_hbm.at[gix.at[j + b + 2]], bufs[b],
                                     gsems[b])

        plsc.subcore_barrier()
        ob = t * OUT_TILE_ROWS

        @pl.when(t < NS - 1)
        def _():
            pltpu.sync_copy(acc.at[pl.ds(ob, OUT_TILE_ROWS)],
                            out_hbm.at[pl.ds(ob, OUT_TILE_ROWS),
                                       pl.ds(q * DQ, DQ)])

        @pl.when(t == NS - 1)
        def _():
            pltpu.sync_copy(acc.at[pl.ds(ob, LAST_TILE_ROWS)],
                            out_hbm.at[pl.ds(ob, LAST_TILE_ROWS),
                                       pl.ds(q * DQ, DQ)])

        plsc.subcore_barrier()


def _tc1_body(x_ref, agg_ref, w1aT, w2aT, w1bT, w2bT,
              b1a, b2a, b1b, b2b, o3, s_ref, ss_ref):
    i = pl.program_id(0)
    x = x_ref[...]
    a = agg_ref[...]
    asum = a[0] + a[1] + a[2]
    s = jnp.zeros((1, D), jnp.float32)
    ss = jnp.zeros((1, D), jnp.float32)
    f32 = jnp.float32
    for r in range(R):
        h1 = x + a[r]
        h2 = x + (asum - a[r])
        g1 = jnp.maximum(jnp.dot(h1, w1aT[...], preferred_element_type=f32)
                         + b1a[...], 0.0)
        v1 = jnp.dot(g1, w2aT[...], preferred_element_type=f32) + b2a[...]
        g2 = jnp.maximum(jnp.dot(h2, w1bT[...], preferred_element_type=f32)
                         + b1b[...], 0.0)
        v2 = jnp.dot(g2, w2bT[...], preferred_element_type=f32) + b2b[...]
        o = v1 + v2
        o3[r] = o
        s = s + jnp.sum(o, axis=0, keepdims=True)
        ss = ss + jnp.sum(o * o, axis=0, keepdims=True)

    @pl.when(i == 0)
    def _():
        s_ref[...] = s
        ss_ref[...] = ss

    @pl.when(i > 0)
    def _():
        s_ref[...] += s
        ss_ref[...] += ss


_tc1 = pl.pallas_call(
    _tc1_body,
    grid=(NBLK,),
    in_specs=[
        pl.BlockSpec((BLK, D), lambda i: (i, 0)),
        pl.BlockSpec((R, BLK, D), lambda i: (0, i, 0)),
        pl.BlockSpec((D, D), lambda i: (0, 0)),
        pl.BlockSpec((D, D), lambda i: (0, 0)),
        pl.BlockSpec((D, D), lambda i: (0, 0)),
        pl.BlockSpec((D, D), lambda i: (0, 0)),
        pl.BlockSpec((1, D), lambda i: (0, 0)),
        pl.BlockSpec((1, D), lambda i: (0, 0)),
        pl.BlockSpec((1, D), lambda i: (0, 0)),
        pl.BlockSpec((1, D), lambda i: (0, 0)),
    ],
    out_specs=[
        pl.BlockSpec((R, BLK, D), lambda i: (0, i, 0)),
        pl.BlockSpec((1, D), lambda i: (0, 0)),
        pl.BlockSpec((1, D), lambda i: (0, 0)),
    ],
    out_shape=[
        jax.ShapeDtypeStruct((R, N, D), jnp.float32),
        jax.ShapeDtypeStruct((1, D), jnp.float32),
        jax.ShapeDtypeStruct((1, D), jnp.float32),
    ],
)


def _tc2_body(o3, s_ref, ss_ref, g_ref, b_ref, out):
    inv = 1.0 / float(R * N)
    mean = s_ref[...] * inv
    var = ss_ref[...] * inv - mean * mean
    scale = g_ref[...] * lax.rsqrt(var + BN_EPS)
    shift = b_ref[...] - mean * scale
    for r in range(R):
        out[r] = o3[r] * scale + shift


_tc2 = pl.pallas_call(
    _tc2_body,
    grid=(NBLK,),
    in_specs=[
        pl.BlockSpec((R, BLK, D), lambda i: (0, i, 0)),
        pl.BlockSpec((1, D), lambda i: (0, 0)),
        pl.BlockSpec((1, D), lambda i: (0, 0)),
        pl.BlockSpec((1, D), lambda i: (0, 0)),
        pl.BlockSpec((1, D), lambda i: (0, 0)),
    ],
    out_specs=pl.BlockSpec((R, BLK, D), lambda i: (0, i, 0)),
    out_shape=jax.ShapeDtypeStruct((R, N, D), jnp.float32),
)


def kernel(vfts, adjs, rels, W1a, b1a, W2a, b2a, W1b, b1b, W2b, b2b, gamma, beta):
    src = adjs[0]
    dst = adjs[1]
    pad = E_PAD - E
    sidx = rels * N + dst
    sidx_p = jnp.concatenate([sidx, jnp.full((pad,), DUMMY, jnp.int32)])
    src_p = jnp.concatenate([src, jnp.zeros((pad,), jnp.int32)])
    base = src_p * NQ
    off = jnp.array([0, 2, 1, 3], jnp.int32)
    gidx = (base[None, :] + off[:, None]).reshape(NC * 2 * NS, CHUNKS, CH)
    sidx_slab = sidx_p.reshape(NS, CHUNKS, CH)
    vq = vfts.reshape(N * NQ, DQ)

    agg = _sc_agg(gidx, sidx_slab, vq)
    agg3 = agg.reshape(R, N, D)

    o3, s, ss = _tc1(
        vfts, agg3, W1a.T, W2a.T, W1b.T, W2b.T,
        b1a.reshape(1, D), b2a.reshape(1, D),
        b1b.reshape(1, D), b2b.reshape(1, D),
    )
    out3 = _tc2(o3, s, ss, gamma.reshape(1, D), beta.reshape(1, D))
    return jnp.transpose(out3, (1, 0, 2))
